# baseline (speedup 1.0000x reference)
import jax
import jax.numpy as jnp
from jax.experimental import pallas as pl
from jax.experimental.pallas import tpu as pltpu


def _ffn_kernel(nf, nd, BF, BD,
                x_ref, cond_ref, w1_ref, b1_ref, w2_ref, b2_ref, out_ref,
                h_ref, a_ref):
    s = pl.program_id(1)

    @pl.when(s == 0)
    def _make_a():
        a_ref[...] = x_ref[0] + cond_ref[0]

    @pl.when(s < nf)
    def _phase1():
        w1 = w1_ref[0].astype(jnp.bfloat16)
        h = jax.lax.dot_general(
            a_ref[...], w1,
            dimension_numbers=(((1,), (1,)), ((), ())),
            preferred_element_type=jnp.float32,
        )
        h = jax.nn.gelu(h + b1_ref[0])
        h_ref[:, pl.ds(s * BF, BF)] = h.astype(jnp.bfloat16)

    @pl.when(s >= nf)
    def _phase2():
        w2 = w2_ref[0].astype(jnp.bfloat16)
        o = jax.lax.dot_general(
            h_ref[...], w2,
            dimension_numbers=(((1,), (1,)), ((), ())),
            preferred_element_type=jnp.float32,
        )
        out_ref[0] = o + b2_ref[0]


def kernel(x, cond, W1, b1, W2, b2):
    E, T, D = x.shape
    F = W1.shape[1]
    BF = min(512, F)
    nf = F // BF
    BD = min(256, D)
    nd = D // BD

    xc = x.astype(jnp.bfloat16)
    cc = cond.astype(jnp.bfloat16)

    b1r = b1.reshape(E * nf, 1, BF)
    b2r = b2.reshape(E * nd, 1, BD)

    ns = nf + nd
    grid = (E, ns)

    def w1_idx(e, s):
        return (e, jnp.minimum(s, nf - 1), 0)

    def b1_idx(e, s):
        return (e * nf + jnp.minimum(s, nf - 1), 0, 0)

    def w2_idx(e, s):
        return (e, jnp.maximum(s - nf, 0), 0)

    def b2_idx(e, s):
        return (e * nd + jnp.maximum(s - nf, 0), 0, 0)

    import functools
    body = functools.partial(_ffn_kernel, nf, nd, BF, BD)

    return pl.pallas_call(
        body,
        grid=grid,
        in_specs=[
            pl.BlockSpec((1, T, D), lambda e, s: (e, 0, 0)),
            pl.BlockSpec((1, T, D), lambda e, s: (e, 0, 0)),
            pl.BlockSpec((1, BF, D), w1_idx),
            pl.BlockSpec((1, 1, BF), b1_idx),
            pl.BlockSpec((1, BD, F), w2_idx),
            pl.BlockSpec((1, 1, BD), b2_idx),
        ],
        out_specs=pl.BlockSpec(
            (1, T, BD), lambda e, s: (e, 0, jnp.maximum(s - nf, 0))
        ),
        out_shape=jax.ShapeDtypeStruct((E, T, D), jnp.float32),
        scratch_shapes=[
            pltpu.VMEM((T, F), jnp.bfloat16),
            pltpu.VMEM((T, D), jnp.bfloat16),
        ],
        compiler_params=pltpu.CompilerParams(
            dimension_semantics=("parallel", "arbitrary")
        ),
    )(xc, cc, W1, b1r, W2, b2r)


# speedup vs baseline: 1.1473x; 1.1473x over previous
import functools

import jax
import jax.numpy as jnp
from jax.experimental import pallas as pl
from jax.experimental.pallas import tpu as pltpu


def _ffn_kernel(E, nf, nd, BF, BD,
                x_hbm, cond_hbm, w1_ref, b1_ref, w2_ref, b2_ref, out_ref,
                xs_ref, cs_ref, a_ref, h_ref, sem_x, sem_c):
    e = pl.program_id(0)
    s = pl.program_id(1)

    @pl.when(jnp.logical_and(e == 0, s == 0))
    def _first_prefetch():
        pltpu.make_async_copy(x_hbm.at[0], xs_ref, sem_x).start()
        pltpu.make_async_copy(cond_hbm.at[0], cs_ref, sem_c).start()

    @pl.when(s == 0)
    def _make_a():
        pltpu.make_async_copy(x_hbm.at[e], xs_ref, sem_x).wait()
        pltpu.make_async_copy(cond_hbm.at[e], cs_ref, sem_c).wait()
        a_ref[...] = (xs_ref[...] + cs_ref[...]).astype(jnp.bfloat16)

    @pl.when(jnp.logical_and(s == 1, e < E - 1))
    def _prefetch_next():
        nxt = jnp.minimum(e + 1, E - 1)
        pltpu.make_async_copy(x_hbm.at[nxt], xs_ref, sem_x).start()
        pltpu.make_async_copy(cond_hbm.at[nxt], cs_ref, sem_c).start()

    @pl.when(s < nf)
    def _phase1():
        w1 = w1_ref[0].astype(jnp.bfloat16)
        h = jax.lax.dot_general(
            a_ref[...], w1,
            dimension_numbers=(((1,), (1,)), ((), ())),
            preferred_element_type=jnp.float32,
        )
        h = jax.nn.gelu(h + b1_ref[0])
        h_ref[:, pl.ds(s * BF, BF)] = h.astype(jnp.bfloat16)

    @pl.when(s >= nf)
    def _phase2():
        w2 = w2_ref[0].astype(jnp.bfloat16)
        o = jax.lax.dot_general(
            h_ref[...], w2,
            dimension_numbers=(((1,), (1,)), ((), ())),
            preferred_element_type=jnp.float32,
        )
        out_ref[0] = o + b2_ref[0]


def kernel(x, cond, W1, b1, W2, b2):
    E, T, D = x.shape
    F = W1.shape[1]
    BF = min(512, F)
    nf = F // BF
    BD = min(256, D)
    nd = D // BD

    b1r = b1.reshape(E * nf, 1, BF)
    b2r = b2.reshape(E * nd, 1, BD)

    ns = nf + nd
    grid = (E, ns)

    def w1_idx(e, s):
        return (e, jnp.minimum(s, nf - 1), 0)

    def b1_idx(e, s):
        return (e * nf + jnp.minimum(s, nf - 1), 0, 0)

    def w2_idx(e, s):
        return (e, jnp.maximum(s - nf, 0), 0)

    def b2_idx(e, s):
        return (e * nd + jnp.maximum(s - nf, 0), 0, 0)

    body = functools.partial(_ffn_kernel, E, nf, nd, BF, BD)

    return pl.pallas_call(
        body,
        grid=grid,
        in_specs=[
            pl.BlockSpec(memory_space=pltpu.MemorySpace.HBM),
            pl.BlockSpec(memory_space=pltpu.MemorySpace.HBM),
            pl.BlockSpec((1, BF, D), w1_idx),
            pl.BlockSpec((1, 1, BF), b1_idx),
            pl.BlockSpec((1, BD, F), w2_idx),
            pl.BlockSpec((1, 1, BD), b2_idx),
        ],
        out_specs=pl.BlockSpec(
            (1, T, BD), lambda e, s: (e, 0, jnp.maximum(s - nf, 0))
        ),
        out_shape=jax.ShapeDtypeStruct((E, T, D), jnp.float32),
        scratch_shapes=[
            pltpu.VMEM((T, D), jnp.float32),
            pltpu.VMEM((T, D), jnp.float32),
            pltpu.VMEM((T, D), jnp.bfloat16),
            pltpu.VMEM((T, F), jnp.bfloat16),
            pltpu.SemaphoreType.DMA,
            pltpu.SemaphoreType.DMA,
        ],
        compiler_params=pltpu.CompilerParams(
            dimension_semantics=("arbitrary", "arbitrary")
        ),
    )(x, cond, W1, b1r, W2, b2r)


# speedup vs baseline: 1.2311x; 1.0731x over previous
import functools

import jax
import jax.numpy as jnp
from jax.experimental import pallas as pl
from jax.experimental.pallas import tpu as pltpu


def _ffn_kernel(E, nf, nd, BF, BD,
                x_hbm, cond_hbm, w1_ref, b1_ref, w2_ref, b2_ref, out_ref,
                xs_ref, cs_ref, a_ref, h_ref, sem_x, sem_c):
    e = pl.program_id(0)
    s = pl.program_id(1)

    @pl.when(jnp.logical_and(e == 0, s == 0))
    def _first_prefetch():
        pltpu.make_async_copy(x_hbm.at[0], xs_ref, sem_x).start()
        pltpu.make_async_copy(cond_hbm.at[0], cs_ref, sem_c).start()

    @pl.when(jnp.logical_and(e < E, s == 0))
    def _make_a():
        pltpu.make_async_copy(x_hbm.at[e], xs_ref, sem_x).wait()
        pltpu.make_async_copy(cond_hbm.at[e], cs_ref, sem_c).wait()
        a_ref[...] = (xs_ref[...] + cs_ref[...]).astype(jnp.bfloat16)

    @pl.when(jnp.logical_and(s == 1, e < E - 1))
    def _prefetch_next():
        nxt = jnp.minimum(e + 1, E - 1)
        pltpu.make_async_copy(x_hbm.at[nxt], xs_ref, sem_x).start()
        pltpu.make_async_copy(cond_hbm.at[nxt], cs_ref, sem_c).start()

    @pl.when(e < E)
    def _phase1():
        w1 = w1_ref[0].astype(jnp.bfloat16)
        h = jax.lax.dot_general(
            a_ref[...], w1,
            dimension_numbers=(((1,), (1,)), ((), ())),
            preferred_element_type=jnp.float32,
        )
        h = jax.nn.gelu(h + b1_ref[0])
        h_ref[e % 2, :, pl.ds(s * BF, BF)] = h.astype(jnp.bfloat16)

    @pl.when(e > 0)
    def _phase2():
        w2 = w2_ref[0].astype(jnp.bfloat16)
        o = jax.lax.dot_general(
            h_ref[(e + 1) % 2], w2,
            dimension_numbers=(((1,), (1,)), ((), ())),
            preferred_element_type=jnp.float32,
        )
        out_ref[0] = o + b2_ref[0]


def kernel(x, cond, W1, b1, W2, b2):
    E, T, D = x.shape
    F = W1.shape[1]
    BF = min(512, F)
    nf = F // BF
    BD = D // nf
    nd = nf

    b1r = b1.reshape(E * nf, 1, BF)
    b2r = b2.reshape(E * nd, 1, BD)

    grid = (E + 1, nf)

    def w1_idx(e, s):
        return (jnp.minimum(e, E - 1), s, 0)

    def b1_idx(e, s):
        return (jnp.minimum(e, E - 1) * nf + s, 0, 0)

    def w2_idx(e, s):
        return (jnp.maximum(e - 1, 0), jnp.where(e == 0, 0, s), 0)

    def b2_idx(e, s):
        return (jnp.maximum(e - 1, 0) * nd + jnp.where(e == 0, 0, s), 0, 0)

    def out_idx(e, s):
        return (jnp.maximum(e - 1, 0), 0, jnp.where(e == 0, 0, s))

    body = functools.partial(_ffn_kernel, E, nf, nd, BF, BD)

    return pl.pallas_call(
        body,
        grid=grid,
        in_specs=[
            pl.BlockSpec(memory_space=pltpu.MemorySpace.HBM),
            pl.BlockSpec(memory_space=pltpu.MemorySpace.HBM),
            pl.BlockSpec((1, BF, D), w1_idx),
            pl.BlockSpec((1, 1, BF), b1_idx),
            pl.BlockSpec((1, BD, F), w2_idx),
            pl.BlockSpec((1, 1, BD), b2_idx),
        ],
        out_specs=pl.BlockSpec((1, T, BD), out_idx),
        out_shape=jax.ShapeDtypeStruct((E, T, D), jnp.float32),
        scratch_shapes=[
            pltpu.VMEM((T, D), jnp.float32),
            pltpu.VMEM((T, D), jnp.float32),
            pltpu.VMEM((T, D), jnp.bfloat16),
            pltpu.VMEM((2, T, F), jnp.bfloat16),
            pltpu.SemaphoreType.DMA,
            pltpu.SemaphoreType.DMA,
        ],
        compiler_params=pltpu.CompilerParams(
            dimension_semantics=("arbitrary", "arbitrary")
        ),
    )(x, cond, W1, b1r, W2, b2r)


# speedup vs baseline: 1.2371x; 1.0049x over previous
import functools

import jax
import jax.numpy as jnp
from jax.experimental import pallas as pl
from jax.experimental.pallas import tpu as pltpu


def _ffn_kernel(E, nf, nd, BF, BD, CT,
                x_hbm, cond_hbm, w1_ref, b1_ref, w2_ref, b2_ref, out_ref,
                xs_ref, cs_ref, a_ref, h_ref, sem_x, sem_c):
    e = pl.program_id(0)
    s = pl.program_id(1)
    nc = 4

    @pl.when(jnp.logical_and(e == 0, s == 0))
    def _bootstrap():
        def body(k, _):
            slot = jax.lax.rem(k, 2)
            cx = pltpu.make_async_copy(
                x_hbm.at[0, pl.ds(k * CT, CT)], xs_ref.at[slot], sem_x)
            cc = pltpu.make_async_copy(
                cond_hbm.at[0, pl.ds(k * CT, CT)], cs_ref.at[slot], sem_c)
            cx.start()
            cc.start()
            cx.wait()
            cc.wait()
            a_ref[0, pl.ds(k * CT, CT)] = (
                xs_ref[slot] + cs_ref[slot]).astype(jnp.bfloat16)
            return 0
        jax.lax.fori_loop(0, nc, body, 0)

    nxt = jnp.minimum(e + 1, E - 1)

    @pl.when(jnp.logical_and(e < E - 1,
                             jnp.logical_and(s >= 1, s <= nc)))
    def _start_chunk():
        k = s - 1
        slot = jax.lax.rem(k, 2)
        pltpu.make_async_copy(
            x_hbm.at[nxt, pl.ds(k * CT, CT)], xs_ref.at[slot], sem_x).start()
        pltpu.make_async_copy(
            cond_hbm.at[nxt, pl.ds(k * CT, CT)], cs_ref.at[slot], sem_c).start()

    @pl.when(jnp.logical_and(e < E - 1,
                             jnp.logical_and(s >= 2, s <= nc + 1)))
    def _fold_chunk():
        k = s - 2
        slot = jax.lax.rem(k, 2)
        pltpu.make_async_copy(
            x_hbm.at[nxt, pl.ds(k * CT, CT)], xs_ref.at[slot], sem_x).wait()
        pltpu.make_async_copy(
            cond_hbm.at[nxt, pl.ds(k * CT, CT)], cs_ref.at[slot], sem_c).wait()
        a_ref[(e + 1) % 2, pl.ds(k * CT, CT)] = (
            xs_ref[slot] + cs_ref[slot]).astype(jnp.bfloat16)

    @pl.when(e < E)
    def _phase1():
        w1 = w1_ref[0].astype(jnp.bfloat16)
        h = jax.lax.dot_general(
            a_ref[e % 2], w1,
            dimension_numbers=(((1,), (1,)), ((), ())),
            preferred_element_type=jnp.float32,
        )
        h = jax.nn.gelu(h + b1_ref[0])
        h_ref[e % 2, :, pl.ds(s * BF, BF)] = h.astype(jnp.bfloat16)

    @pl.when(e > 0)
    def _phase2():
        w2 = w2_ref[0].astype(jnp.bfloat16)
        o = jax.lax.dot_general(
            h_ref[(e + 1) % 2], w2,
            dimension_numbers=(((1,), (1,)), ((), ())),
            preferred_element_type=jnp.float32,
        )
        out_ref[0] = o + b2_ref[0]


def kernel(x, cond, W1, b1, W2, b2):
    E, T, D = x.shape
    F = W1.shape[1]
    BF = min(512, F)
    nf = F // BF
    BD = D // nf
    nd = nf
    CT = T // 4

    b1r = b1.reshape(E * nf, 1, BF)
    b2r = b2.reshape(E * nd, 1, BD)

    grid = (E + 1, nf)

    def w1_idx(e, s):
        return (jnp.minimum(e, E - 1), s, 0)

    def b1_idx(e, s):
        return (jnp.minimum(e, E - 1) * nf + s, 0, 0)

    def w2_idx(e, s):
        return (jnp.maximum(e - 1, 0), jnp.where(e == 0, 0, s), 0)

    def b2_idx(e, s):
        return (jnp.maximum(e - 1, 0) * nd + jnp.where(e == 0, 0, s), 0, 0)

    def out_idx(e, s):
        return (jnp.maximum(e - 1, 0), 0, jnp.where(e == 0, 0, s))

    body = functools.partial(_ffn_kernel, E, nf, nd, BF, BD, CT)

    return pl.pallas_call(
        body,
        grid=grid,
        in_specs=[
            pl.BlockSpec(memory_space=pltpu.MemorySpace.HBM),
            pl.BlockSpec(memory_space=pltpu.MemorySpace.HBM),
            pl.BlockSpec((1, BF, D), w1_idx),
            pl.BlockSpec((1, 1, BF), b1_idx),
            pl.BlockSpec((1, BD, F), w2_idx),
            pl.BlockSpec((1, 1, BD), b2_idx),
        ],
        out_specs=pl.BlockSpec((1, T, BD), out_idx),
        out_shape=jax.ShapeDtypeStruct((E, T, D), jnp.float32),
        scratch_shapes=[
            pltpu.VMEM((2, T // 4, D), jnp.float32),
            pltpu.VMEM((2, T // 4, D), jnp.float32),
            pltpu.VMEM((2, T, D), jnp.bfloat16),
            pltpu.VMEM((2, T, F), jnp.bfloat16),
            pltpu.SemaphoreType.DMA,
            pltpu.SemaphoreType.DMA,
        ],
        compiler_params=pltpu.CompilerParams(
            dimension_semantics=("arbitrary", "arbitrary")
        ),
    )(x, cond, W1, b1r, W2, b2r)
